# Initial kernel scaffold; baseline (speedup 1.0000x reference)
import jax
import jax.numpy as jnp
from jax.experimental import pallas as pl


def kernel(x, edge_index, params):
    raise NotImplementedError("write your pallas kernel here")


# speedup vs baseline: 4.4089x; 4.4089x over previous
import functools
import math

import jax
import jax.numpy as jnp
from jax import lax
from jax.experimental import pallas as pl
from jax.experimental.pallas import tpu as pltpu
from jax.experimental.pallas import tpu_sc as plsc

N = 10000
E = 320000
D = 128
NC = 2
NS = 16
NW = NC * NS
EPT = E // NW
CHUNK = 80
NCHUNK = EPT // CHUNK
RPS = 624
TAIL = N - NS * RPS

_INV = 1.0 / math.sqrt(1.0 + 1e-5)


def _segment_sum_sc(h, src, dst, zeros):
    mesh = plsc.VectorSubcoreMesh(
        core_axis_name="c", subcore_axis_name="s", num_cores=NC, num_subcores=NS
    )

    @functools.partial(
        pl.kernel,
        out_type=jax.ShapeDtypeStruct((NC, N, D), jnp.float32),
        mesh=mesh,
        scratch_types=[
            pltpu.VMEM((CHUNK,), jnp.int32),
            pltpu.VMEM((CHUNK,), jnp.int32),
            pltpu.VMEM((CHUNK, D), jnp.float32),
            pltpu.VMEM_SHARED((N, D), jnp.float32),
            pltpu.SemaphoreType.DMA,
        ],
    )
    def k(h_hbm, src_hbm, dst_hbm, z_hbm, out_hbm, srcv, dstv, rows, acc, sem):
        cid = lax.axis_index("c")
        sid = lax.axis_index("s")
        wid = sid * NC + cid
        r0 = sid * RPS

        pltpu.sync_copy(z_hbm.at[pl.ds(r0, RPS)], acc.at[pl.ds(r0, RPS)])

        @pl.when(sid == NS - 1)
        def _():
            pltpu.sync_copy(z_hbm.at[pl.ds(NS * RPS, TAIL)],
                            acc.at[pl.ds(NS * RPS, TAIL)])

        plsc.subcore_barrier()

        base = wid * EPT

        @pl.loop(0, NCHUNK)
        def _(i):
            off = base + i * CHUNK
            pltpu.sync_copy(src_hbm.at[pl.ds(off, CHUNK)], srcv)
            pltpu.sync_copy(dst_hbm.at[pl.ds(off, CHUNK)], dstv)
            pltpu.async_copy(h_hbm.at[srcv], rows, sem).wait()
            pltpu.sync_copy(rows, acc.at[dstv], add=True)

        plsc.subcore_barrier()
        pltpu.sync_copy(acc.at[pl.ds(r0, RPS)], out_hbm.at[cid].at[pl.ds(r0, RPS)])

        @pl.when(sid == NS - 1)
        def _():
            pltpu.sync_copy(acc.at[pl.ds(NS * RPS, TAIL)],
                            out_hbm.at[cid].at[pl.ds(NS * RPS, TAIL)])

    return k(h, src, dst, zeros)


def _mlp0_body(x_ref, p0_ref, p1_ref, w1t_ref, b1_ref, g1_ref, be1_ref,
               w2t_ref, b2_ref, g0_ref, be0_ref, o_ref):
    t = x_ref[...] + p0_ref[...] + p1_ref[...]
    a = jnp.dot(t, w1t_ref[...], preferred_element_type=jnp.float32) + b1_ref[...]
    a = jnp.maximum(a * (_INV * g1_ref[...]) + be1_ref[...], 0.0)
    h = jnp.dot(a, w2t_ref[...], preferred_element_type=jnp.float32) + b2_ref[...]
    o_ref[...] = jnp.maximum(h * (_INV * g0_ref[...]) + be0_ref[...], 0.0)


def _head_body(h_ref, p0_ref, p1_ref, w1t_ref, b1_ref, g1_ref, be1_ref,
               w2t_ref, b2_ref, wct_ref, bc_ref, wst_ref, bs_ref,
               wmt_ref, bm_ref, main_ref, sim_ref, he_ref):
    t = h_ref[...] + p0_ref[...] + p1_ref[...]
    a = jnp.dot(t, w1t_ref[...], preferred_element_type=jnp.float32) + b1_ref[...]
    a = jnp.maximum(a * (_INV * g1_ref[...]) + be1_ref[...], 0.0)
    h2 = jnp.dot(a, w2t_ref[...], preferred_element_type=jnp.float32) + b2_ref[...]

    main = jnp.dot(h2, wct_ref[...], preferred_element_type=jnp.float32) + bc_ref[...]
    m = jnp.max(main, axis=-1, keepdims=True)
    s = main - m
    main_ref[...] = s - jnp.log(jnp.sum(jnp.exp(s), axis=-1, keepdims=True))

    sim = jnp.dot(h2, wst_ref[...], preferred_element_type=jnp.float32) + bs_ref[...]
    ms = jnp.max(sim, axis=-1, keepdims=True)
    es = jnp.exp(sim - ms)
    sim_ref[...] = es / jnp.sum(es, axis=-1, keepdims=True)

    he = jnp.dot(h2, wmt_ref[...], preferred_element_type=jnp.float32) + bm_ref[...]
    he_ref[...] = 1.0 / (1.0 + jnp.exp(-he))


_BM = 1000


def _row(i):
    return (i, 0)


def _fixed(i):
    return (0, 0)


def _mlp0(x, p0, p1, w1t, b1, g1, be1, w2t, b2, g0, be0):
    rspec = pl.BlockSpec((_BM, D), _row)
    wspec = pl.BlockSpec((D, D), _fixed)
    vspec = pl.BlockSpec((1, D), _fixed)
    return pl.pallas_call(
        _mlp0_body,
        out_shape=jax.ShapeDtypeStruct((N, D), jnp.float32),
        grid=(N // _BM,),
        in_specs=[rspec, rspec, rspec, wspec, vspec, vspec, vspec,
                  wspec, vspec, vspec, vspec],
        out_specs=rspec,
    )(x, p0, p1, w1t, b1, g1, be1, w2t, b2, g0, be0)


def _heads(h, p0, p1, w1t, b1, g1, be1, w2t, b2, wct, bc, wst, bs, wmt, bm):
    rspec = pl.BlockSpec((_BM, D), _row)
    wspec = pl.BlockSpec((D, D), _fixed)
    vspec = pl.BlockSpec((1, D), _fixed)
    return pl.pallas_call(
        _head_body,
        out_shape=(
            jax.ShapeDtypeStruct((N, 40), jnp.float32),
            jax.ShapeDtypeStruct((N, 40), jnp.float32),
            jax.ShapeDtypeStruct((N, 2), jnp.float32),
        ),
        grid=(N // _BM,),
        in_specs=[rspec, rspec, rspec, wspec, vspec, vspec, vspec,
                  wspec, vspec,
                  pl.BlockSpec((D, 40), _fixed), pl.BlockSpec((1, 40), _fixed),
                  pl.BlockSpec((D, 40), _fixed), pl.BlockSpec((1, 40), _fixed),
                  pl.BlockSpec((D, 2), _fixed), pl.BlockSpec((1, 2), _fixed)],
        out_specs=(
            pl.BlockSpec((_BM, 40), _row),
            pl.BlockSpec((_BM, 40), _row),
            pl.BlockSpec((_BM, 2), _row),
        ),
    )(h, p0, p1, w1t, b1, g1, be1, w2t, b2, wct, bc, wst, bs, wmt, bm)


def kernel(x, edge_index, params):
    src = edge_index[0].astype(jnp.int32)
    dst = edge_index[1].astype(jnp.int32)
    zeros = jnp.zeros((N, D), jnp.float32)

    c0, c1 = params["conv0"], params["conv1"]

    def vec(v):
        return v.reshape(1, -1)

    parts0 = _segment_sum_sc(x, src, dst, zeros)
    h1 = _mlp0(
        x, parts0[0], parts0[1],
        c0["lin1"]["W"].T, vec(c0["lin1"]["b"]), vec(c0["bn"]["g"]), vec(c0["bn"]["be"]),
        c0["lin2"]["W"].T, vec(c0["lin2"]["b"]),
        vec(params["bn0"]["g"]), vec(params["bn0"]["be"]),
    )

    parts1 = _segment_sum_sc(h1, src, dst, zeros)
    wmt = jnp.concatenate([params["homo"]["W"].T, params["ent"]["W"].T], axis=1)
    bm = jnp.concatenate([params["homo"]["b"], params["ent"]["b"]]).reshape(1, 2)
    main, sim, he = _heads(
        h1, parts1[0], parts1[1],
        c1["lin1"]["W"].T, vec(c1["lin1"]["b"]), vec(c1["bn"]["g"]), vec(c1["bn"]["be"]),
        c1["lin2"]["W"].T, vec(c1["lin2"]["b"]),
        params["cls"]["W"].T, vec(params["cls"]["b"]),
        params["sim"]["W"].T, vec(params["sim"]["b"]),
        wmt, bm,
    )
    return main, sim, he[:, 0], he[:, 1]
